# Initial kernel scaffold; baseline (speedup 1.0000x reference)
import jax
import jax.numpy as jnp
from jax.experimental import pallas as pl


def kernel(input_ids, embedding_table):
    raise NotImplementedError("write your pallas kernel here")


# speedup vs baseline: 1.4843x; 1.4843x over previous
import functools

import jax
import jax.numpy as jnp
from jax import lax
from jax.experimental import pallas as pl
from jax.experimental.pallas import tpu as pltpu
from jax.experimental.pallas import tpu_sc as plsc

_CHUNK = 128
_NBUF = 4


@functools.lru_cache(maxsize=None)
def _build(n_tokens: int, vocab: int, embed: int):
    info = plsc.get_sparse_core_info()
    nw = info.num_cores * info.num_subcores
    assert n_tokens % (nw * _CHUNK) == 0
    per_w = n_tokens // nw
    nchunk = per_w // _CHUNK
    assert nchunk % _NBUF == 0
    nc = info.num_cores

    mesh = plsc.VectorSubcoreMesh(core_axis_name="c", subcore_axis_name="s")

    @functools.partial(
        pl.kernel,
        out_type=jax.ShapeDtypeStruct((n_tokens, embed), jnp.float32),
        mesh=mesh,
        scratch_types=[
            pltpu.VMEM((nchunk, _CHUNK), jnp.int32),
            pltpu.VMEM((_NBUF, _CHUNK, embed), jnp.float32),
            pltpu.SemaphoreType.DMA((_NBUF,)),
            pltpu.SemaphoreType.DMA((_NBUF,)),
        ],
        compiler_params=pltpu.CompilerParams(use_tc_tiling_on_sc=False),
    )
    def gather_kernel(ids_hbm, table_hbm, out_hbm, idx_v, rows_v, gsem, ssem):
        wid = lax.axis_index("s") * nc + lax.axis_index("c")
        row0 = wid * per_w

        pltpu.sync_copy(ids_hbm.at[pl.ds(wid * nchunk, nchunk)], idx_v)

        def gather(buf, chunk):
            return pltpu.make_async_copy(
                table_hbm.at[idx_v.at[chunk]], rows_v.at[buf], gsem.at[buf])

        def scatter(buf, chunk):
            return pltpu.make_async_copy(
                rows_v.at[buf],
                out_hbm.at[pl.ds(row0 + chunk * _CHUNK, _CHUNK)],
                ssem.at[buf])

        for b in range(_NBUF):
            gather(b, b).start()

        @pl.loop(0, nchunk, step=_NBUF)
        def _(j):
            for b in range(_NBUF):
                gather(b, j + b).wait()
                scatter(b, j + b).start()
            for b in range(_NBUF):
                scatter(b, j + b).wait()

                @pl.when(j + b + _NBUF < nchunk)
                def _():
                    gather(b, j + b + _NBUF).start()

    return gather_kernel


def kernel(input_ids, embedding_table):
    b, l = input_ids.shape
    vocab, embed = embedding_table.shape
    n = b * l
    ids_flat = input_ids.reshape(n // _CHUNK, _CHUNK)
    fn = _build(n, vocab, embed)
    out = fn(ids_flat, embedding_table)
    return out.reshape(b, l, embed)


# speedup vs baseline: 1.5002x; 1.0107x over previous
import functools

import jax
import jax.numpy as jnp
from jax import lax
from jax.experimental import pallas as pl
from jax.experimental.pallas import tpu as pltpu
from jax.experimental.pallas import tpu_sc as plsc

_CHUNK = 512
_NBUF = 5


@functools.lru_cache(maxsize=None)
def _build(n_tokens: int, vocab: int, embed: int):
    info = plsc.get_sparse_core_info()
    nw = info.num_cores * info.num_subcores
    assert n_tokens % (nw * _CHUNK) == 0
    per_w = n_tokens // nw
    nchunk = per_w // _CHUNK
    assert nchunk % _NBUF == 0
    nc = info.num_cores

    mesh = plsc.VectorSubcoreMesh(core_axis_name="c", subcore_axis_name="s")

    @functools.partial(
        pl.kernel,
        out_type=jax.ShapeDtypeStruct((n_tokens, embed), jnp.float32),
        mesh=mesh,
        scratch_types=[
            pltpu.VMEM((nchunk, _CHUNK), jnp.int32),
            pltpu.VMEM((_NBUF, _CHUNK, embed), jnp.float32),
            pltpu.SemaphoreType.DMA((_NBUF,)),
            pltpu.SemaphoreType.DMA((_NBUF,)),
        ],
        compiler_params=pltpu.CompilerParams(use_tc_tiling_on_sc=False),
    )
    def gather_kernel(ids_hbm, table_hbm, out_hbm, idx_v, rows_v, gsem, ssem):
        wid = lax.axis_index("s") * nc + lax.axis_index("c")
        row0 = wid * per_w

        pltpu.sync_copy(ids_hbm.at[pl.ds(wid * nchunk, nchunk)], idx_v)

        def gather(buf, chunk):
            return pltpu.make_async_copy(
                table_hbm.at[idx_v.at[chunk]], rows_v.at[buf], gsem.at[buf])

        def scatter(buf, chunk):
            return pltpu.make_async_copy(
                rows_v.at[buf],
                out_hbm.at[pl.ds(row0 + chunk * _CHUNK, _CHUNK)],
                ssem.at[buf])

        for b in range(_NBUF):
            gather(b, b).start()

        @pl.loop(0, nchunk, step=_NBUF)
        def _(j):
            for b in range(_NBUF):
                gather(b, j + b).wait()
                scatter(b, j + b).start()
            for b in range(_NBUF):
                scatter(b, j + b).wait()

                @pl.when(j + b + _NBUF < nchunk)
                def _():
                    gather(b, j + b + _NBUF).start()

    return gather_kernel


def kernel(input_ids, embedding_table):
    b, l = input_ids.shape
    vocab, embed = embedding_table.shape
    n = b * l
    ids_flat = input_ids.reshape(n // _CHUNK, _CHUNK)
    fn = _build(n, vocab, embed)
    out = fn(ids_flat, embedding_table)
    return out.reshape(b, l, embed)
